# Initial kernel scaffold; baseline (speedup 1.0000x reference)
import jax
import jax.numpy as jnp
from jax.experimental import pallas as pl


def kernel(feat, proposals, conv1_w, conv1_b, bn1_g, bn1_b, conv2_w, conv2_b, bn2_g, bn2_b, conv3_w, conv3_b, bn3_g, bn3_b, fc_w, fc_b, fcbn_g, fcbn_b, iou_w, iou_b):
    raise NotImplementedError("write your pallas kernel here")


# speedup vs baseline: 2.8423x; 2.8423x over previous
import jax
import jax.numpy as jnp
from jax import lax
from jax.experimental import pallas as pl
from jax.experimental.pallas import tpu as pltpu

DIM = 256
H = W = 36
HW = H * W
POOL = 4
SCALE = 20.0
EPS_BN = 1e-5


def _make_conv_kernel(act):
    def body(x_ref, w_ref, b_ref, *rest):
        if act:
            s_ref, t_ref, y_ref, p_ref = rest
        else:
            y_ref, p_ref = rest
        x = x_ref[0]
        if act:
            x = jnp.maximum(x * s_ref[...] + t_ref[...], 0.0)
        wcol = lax.rem(lax.broadcasted_iota(jnp.int32, (HW, 1), 0), W)
        xm = {
            -1: jnp.where(wcol == W - 1, 0.0, x),
            0: x,
            1: jnp.where(wcol == 0, 0.0, x),
        }
        acc = jnp.zeros((HW, DIM), jnp.float32)
        ti = 0
        for dy in (-1, 0, 1):
            for dx in (-1, 0, 1):
                o = dy * W + dx
                src = xm[dx]
                if o > 0:
                    xs = jnp.concatenate(
                        [src[o:], jnp.zeros((o, DIM), jnp.float32)], axis=0)
                elif o < 0:
                    xs = jnp.concatenate(
                        [jnp.zeros((-o, DIM), jnp.float32), src[:HW + o]], axis=0)
                else:
                    xs = src
                acc = acc + jnp.dot(xs, w_ref[ti],
                                    preferred_element_type=jnp.float32)
                ti += 1
        acc = acc + b_ref[...]
        y_ref[0] = acc
        p_ref[0] = jnp.concatenate(
            [jnp.sum(acc, axis=0, keepdims=True),
             jnp.sum(acc * acc, axis=0, keepdims=True),
             jnp.zeros((6, DIM), jnp.float32)], axis=0)
    return body


def _conv_bn(x, taps, b, s=None, t=None):
    n = x.shape[0]
    act = s is not None
    inputs = [x, taps, b.reshape(1, DIM)]
    in_specs = [
        pl.BlockSpec((1, HW, DIM), lambda i: (i, 0, 0)),
        pl.BlockSpec((9, DIM, DIM), lambda i: (0, 0, 0)),
        pl.BlockSpec((1, DIM), lambda i: (0, 0)),
    ]
    if act:
        inputs += [s.reshape(1, DIM), t.reshape(1, DIM)]
        in_specs += [pl.BlockSpec((1, DIM), lambda i: (0, 0)),
                     pl.BlockSpec((1, DIM), lambda i: (0, 0))]
    return pl.pallas_call(
        _make_conv_kernel(act),
        grid=(n,),
        in_specs=in_specs,
        out_specs=[pl.BlockSpec((1, HW, DIM), lambda i: (i, 0, 0)),
                   pl.BlockSpec((1, 8, DIM), lambda i: (i, 0, 0))],
        out_shape=[jax.ShapeDtypeStruct((n, HW, DIM), jnp.float32),
                   jax.ShapeDtypeStruct((n, 8, DIM), jnp.float32)],
        compiler_params=pltpu.CompilerParams(
            dimension_semantics=("parallel",)),
        name="conv_bn",
    )(*inputs)


def _bn_affine(p, g, beta, n):
    cnt = jnp.float32(n * HW)
    m = jnp.sum(p[:, 0, :], axis=0) / cnt
    ex2 = jnp.sum(p[:, 1, :], axis=0) / cnt
    v = ex2 - m * m
    s = g / jnp.sqrt(v + EPS_BN)
    return s, beta - m * s


def _corner_weights(coord, lo_max):
    k = jnp.clip(jnp.floor(coord), 0.0, lo_max)
    sfrac = coord - k
    p1 = 0.5 * sfrac * sfrac
    p0 = sfrac - p1
    k2 = k[:, :, None]
    p02 = p0[:, :, None]
    p12 = p1[:, :, None]
    npp = coord.shape[0]
    wio = lax.broadcasted_iota(
        jnp.int32, (npp, POOL + 1, W), 2).astype(jnp.float32)
    tri = jnp.where(
        k2 > 0,
        jnp.where(wio < k2, jnp.where(wio == 0, 0.5, 1.0),
                  jnp.where(wio == k2, 0.5, 0.0)),
        0.0)
    return (tri + jnp.where(wio == k2, p02, 0.0)
            + jnp.where(wio == k2 + 1, p12, 0.0))


def _make_pool_kernel(npp):
    nr = npp * POOL

    def body(f_ref, prop_ref, s_ref, t_ref, o_ref):
        fmap = jnp.maximum(f_ref[0] * s_ref[...] + t_ref[...], 0.0)
        p = prop_ref[0]
        x1 = p[:, 0:1] * SCALE
        y1 = p[:, 1:2] * SCALE
        bw = p[:, 2:3] * (SCALE / POOL)
        bh = p[:, 3:4] * (SCALE / POOL)
        gridv = lax.broadcasted_iota(
            jnp.int32, (1, POOL + 1), 1).astype(jnp.float32)
        xs = jnp.clip(x1 + bw * gridv, 0.0, W - 1.0)
        ys = jnp.clip(y1 + bh * gridv, 0.0, H - 1.0)
        hx = _corner_weights(xs, W - 2.0)
        hy = _corner_weights(ys, H - 2.0)
        cx = hx[:, 1:] - hx[:, :-1]
        cy = hy[:, 1:] - hy[:, :-1]
        area = bw * bh
        inva = jnp.where(area > 1e-8, 1.0 / jnp.maximum(area, 1e-8), 0.0)
        cx = cx * inva[:, :, None]
        cyf = jnp.transpose(cy, (1, 0, 2)).reshape(nr, H)
        pm = jnp.dot(cyf, fmap, preferred_element_type=jnp.float32)
        pm2 = pm.reshape(nr * W, DIM)
        rr = lax.broadcasted_iota(jnp.int32, (npp, 1, npp, 1), 0)
        rc = lax.broadcasted_iota(jnp.int32, (npp, 1, npp, 1), 2)
        eye = (rr == rc).astype(jnp.float32)
        cbd = (eye * cx[:, :, None, :]).reshape(nr, npp * W)
        outs = [jnp.dot(cbd, pm2[j * npp * W:(j + 1) * npp * W],
                        preferred_element_type=jnp.float32)
                for j in range(POOL)]
        o_ref[...] = jnp.concatenate(
            [o[None] for o in outs], axis=0).reshape(POOL, 1, npp, POOL, DIM)
    return body


def _pool(fmap, props, st, tt):
    n, npp = props.shape[0], props.shape[1]
    return pl.pallas_call(
        _make_pool_kernel(npp),
        grid=(n,),
        in_specs=[
            pl.BlockSpec((1, H, W * DIM), lambda i: (i, 0, 0)),
            pl.BlockSpec((1, npp, 4), lambda i: (i, 0, 0)),
            pl.BlockSpec((1, W * DIM), lambda i: (0, 0)),
            pl.BlockSpec((1, W * DIM), lambda i: (0, 0)),
        ],
        out_specs=pl.BlockSpec((POOL, 1, npp, POOL, DIM),
                               lambda i: (0, i, 0, 0, 0)),
        out_shape=jax.ShapeDtypeStruct((POOL, n, npp, POOL, DIM), jnp.float32),
        compiler_params=pltpu.CompilerParams(
            dimension_semantics=("parallel",)),
        name="prroi_pool",
    )(fmap, props, st, tt)


def _make_fc_kernel(nrois):
    def body(q_ref, w_ref, b_ref, g_ref, beta_ref, iw_ref, ib_ref, o_ref):
        fcx = jnp.zeros((nrois, DIM), jnp.float32)
        for j in range(POOL):
            qj = q_ref[j].reshape(nrois, POOL * DIM)
            fcx = fcx + jnp.dot(qj, w_ref[j], preferred_element_type=jnp.float32)
        fcx = fcx + b_ref[...]
        m = jnp.mean(fcx, axis=0, keepdims=True)
        v = jnp.mean(fcx * fcx, axis=0, keepdims=True) - m * m
        x = (fcx - m) / jnp.sqrt(v + EPS_BN) * g_ref[...] + beta_ref[...]
        x = jnp.maximum(x, 0.0)
        o_ref[...] = jnp.dot(x, iw_ref[...],
                             preferred_element_type=jnp.float32) + ib_ref[...]
    return body


def _fc_head(q, fcw, fcb, g, beta, iw, ib):
    nrois = q.shape[1] * q.shape[2]
    return pl.pallas_call(
        _make_fc_kernel(nrois),
        out_shape=jax.ShapeDtypeStruct((nrois, 1), jnp.float32),
        name="fc_iou_head",
    )(q.reshape(POOL, nrois, POOL, DIM), fcw, fcb.reshape(1, DIM),
      g.reshape(1, DIM), beta.reshape(1, DIM), iw, ib.reshape(1, 1))


def kernel(feat, proposals, conv1_w, conv1_b, bn1_g, bn1_b, conv2_w, conv2_b,
           bn2_g, bn2_b, conv3_w, conv3_b, bn3_g, bn3_b, fc_w, fc_b,
           fcbn_g, fcbn_b, iou_w, iou_b):
    ni, ns, npp = proposals.shape[0], proposals.shape[1], proposals.shape[2]
    n = ni * ns

    x0 = feat.reshape(n, DIM, HW).transpose(0, 2, 1)
    taps1 = conv1_w.transpose(2, 3, 1, 0).reshape(9, DIM, DIM)
    taps2 = conv2_w.transpose(2, 3, 1, 0).reshape(9, DIM, DIM)
    taps3 = conv3_w.transpose(2, 3, 1, 0).reshape(9, DIM, DIM)

    y1, p1 = _conv_bn(x0, taps1, conv1_b)
    s1, t1 = _bn_affine(p1, bn1_g, bn1_b, n)
    y2, p2 = _conv_bn(y1, taps2, conv2_b, s1, t1)
    s2, t2 = _bn_affine(p2, bn2_g, bn2_b, n)
    y3, p3 = _conv_bn(y2, taps3, conv3_b, s2, t2)
    s3, t3 = _bn_affine(p3, bn3_g, bn3_b, n)

    fmap = y3.reshape(n, H, W * DIM)
    q = _pool(fmap, proposals.reshape(n, npp, 4),
              jnp.tile(s3, W).reshape(1, W * DIM),
              jnp.tile(t3, W).reshape(1, W * DIM))

    fcw = fc_w.reshape(DIM, DIM, POOL, POOL).transpose(2, 3, 1, 0)
    fcw = fcw.reshape(POOL, POOL * DIM, DIM)
    iou = _fc_head(q, fcw, fc_b, fcbn_g, fcbn_b, iou_w.T, iou_b)
    return iou.reshape(ni, ns, npp)


# speedup vs baseline: 3.0055x; 1.0574x over previous
import jax
import jax.numpy as jnp
from jax import lax
from jax.experimental import pallas as pl
from jax.experimental.pallas import tpu as pltpu

DIM = 256
H = W = 36
HW = H * W
POOL = 4
SCALE = 20.0
EPS_BN = 1e-5


def _shift_rows(a, o, rows, cols):
    if o > 0:
        return jnp.concatenate(
            [a[o:], jnp.zeros((o, cols), jnp.float32)], axis=0)
    if o < 0:
        return jnp.concatenate(
            [jnp.zeros((-o, cols), jnp.float32), a[:rows + o]], axis=0)
    return a


def _make_conv_kernel(act):
    def body(x_ref, w_ref, b_ref, *rest):
        if act:
            s_ref, t_ref, y_ref, p_ref = rest
        else:
            y_ref, p_ref = rest
        x = x_ref[0]
        if act:
            x = jnp.maximum(x * s_ref[...] + t_ref[...], 0.0)
        xcat = jnp.concatenate(
            [_shift_rows(x, -W, HW, DIM), x, _shift_rows(x, W, HW, DIM)],
            axis=1)
        z = [jnp.dot(xcat, w_ref[d], preferred_element_type=jnp.float32)
             for d in range(3)]
        wcol = lax.rem(lax.broadcasted_iota(jnp.int32, (HW, 1), 0), W)
        acc = (z[1]
               + jnp.where(wcol == 0, 0.0, _shift_rows(z[0], -1, HW, DIM))
               + jnp.where(wcol == W - 1, 0.0, _shift_rows(z[2], 1, HW, DIM))
               + b_ref[...])
        y_ref[0] = acc
        p_ref[0] = jnp.concatenate(
            [jnp.sum(acc, axis=0, keepdims=True),
             jnp.sum(acc * acc, axis=0, keepdims=True),
             jnp.zeros((6, DIM), jnp.float32)], axis=0)
    return body


def _conv_bn(x, taps, b, s=None, t=None):
    n = x.shape[0]
    act = s is not None
    inputs = [x, taps, b.reshape(1, DIM)]
    in_specs = [
        pl.BlockSpec((1, HW, DIM), lambda i: (i, 0, 0)),
        pl.BlockSpec((3, 3 * DIM, DIM), lambda i: (0, 0, 0)),
        pl.BlockSpec((1, DIM), lambda i: (0, 0)),
    ]
    if act:
        inputs += [s.reshape(1, DIM), t.reshape(1, DIM)]
        in_specs += [pl.BlockSpec((1, DIM), lambda i: (0, 0)),
                     pl.BlockSpec((1, DIM), lambda i: (0, 0))]
    return pl.pallas_call(
        _make_conv_kernel(act),
        grid=(n,),
        in_specs=in_specs,
        out_specs=[pl.BlockSpec((1, HW, DIM), lambda i: (i, 0, 0)),
                   pl.BlockSpec((1, 8, DIM), lambda i: (i, 0, 0))],
        out_shape=[jax.ShapeDtypeStruct((n, HW, DIM), jnp.float32),
                   jax.ShapeDtypeStruct((n, 8, DIM), jnp.float32)],
        compiler_params=pltpu.CompilerParams(
            dimension_semantics=("parallel",)),
        name="conv_bn",
    )(*inputs)


def _bn_affine(p, g, beta, n):
    cnt = jnp.float32(n * HW)
    m = jnp.sum(p[:, 0, :], axis=0) / cnt
    ex2 = jnp.sum(p[:, 1, :], axis=0) / cnt
    v = ex2 - m * m
    s = g / jnp.sqrt(v + EPS_BN)
    return s, beta - m * s


def _corner_weights(coord, lo_max):
    k = jnp.clip(jnp.floor(coord), 0.0, lo_max)
    sfrac = coord - k
    p1 = 0.5 * sfrac * sfrac
    p0 = sfrac - p1
    k2 = k[:, :, None]
    p02 = p0[:, :, None]
    p12 = p1[:, :, None]
    npp = coord.shape[0]
    wio = lax.broadcasted_iota(
        jnp.int32, (npp, POOL + 1, W), 2).astype(jnp.float32)
    tri = jnp.where(
        k2 > 0,
        jnp.where(wio < k2, jnp.where(wio == 0, 0.5, 1.0),
                  jnp.where(wio == k2, 0.5, 0.0)),
        0.0)
    return (tri + jnp.where(wio == k2, p02, 0.0)
            + jnp.where(wio == k2 + 1, p12, 0.0))


def _make_pool_kernel(npp):
    def body(f_ref, prop_ref, s_ref, t_ref, o_ref):
        fmap = jnp.maximum(f_ref[0] * s_ref[...] + t_ref[...], 0.0)
        p = prop_ref[0]
        x1 = p[:, 0:1] * SCALE
        y1 = p[:, 1:2] * SCALE
        bw = p[:, 2:3] * (SCALE / POOL)
        bh = p[:, 3:4] * (SCALE / POOL)
        gridv = lax.broadcasted_iota(
            jnp.int32, (1, POOL + 1), 1).astype(jnp.float32)
        xs = jnp.clip(x1 + bw * gridv, 0.0, W - 1.0)
        ys = jnp.clip(y1 + bh * gridv, 0.0, H - 1.0)
        hx = _corner_weights(xs, W - 2.0)
        hy = _corner_weights(ys, H - 2.0)
        cx = hx[:, 1:] - hx[:, :-1]
        cy = hy[:, 1:] - hy[:, :-1]
        area = bw * bh
        inva = jnp.where(area > 1e-8, 1.0 / jnp.maximum(area, 1e-8), 0.0)
        cx = cx * inva[:, :, None]
        wt = (cx[:, :, None, None, :]
              * cy[:, None, :, :, None]).reshape(npp * POOL * POOL, HW)
        pooled = jnp.dot(wt, fmap, preferred_element_type=jnp.float32)
        o_ref[0] = pooled.reshape(npp, POOL * POOL * DIM)
    return body


def _pool(fmap, props, s3, t3):
    n, npp = props.shape[0], props.shape[1]
    return pl.pallas_call(
        _make_pool_kernel(npp),
        grid=(n,),
        in_specs=[
            pl.BlockSpec((1, HW, DIM), lambda i: (i, 0, 0)),
            pl.BlockSpec((1, npp, 4), lambda i: (i, 0, 0)),
            pl.BlockSpec((1, DIM), lambda i: (0, 0)),
            pl.BlockSpec((1, DIM), lambda i: (0, 0)),
        ],
        out_specs=pl.BlockSpec((1, npp, POOL * POOL * DIM),
                               lambda i: (i, 0, 0)),
        out_shape=jax.ShapeDtypeStruct((n, npp, POOL * POOL * DIM),
                                       jnp.float32),
        compiler_params=pltpu.CompilerParams(
            dimension_semantics=("parallel",)),
        name="prroi_pool",
    )(fmap, props, s3.reshape(1, DIM), t3.reshape(1, DIM))


def _make_fc_kernel(nrois):
    def body(q_ref, w_ref, b_ref, g_ref, beta_ref, iw_ref, ib_ref, o_ref):
        fcx = jnp.dot(q_ref[...], w_ref[...],
                      preferred_element_type=jnp.float32) + b_ref[...]
        m = jnp.mean(fcx, axis=0, keepdims=True)
        v = jnp.mean(fcx * fcx, axis=0, keepdims=True) - m * m
        x = (fcx - m) / jnp.sqrt(v + EPS_BN) * g_ref[...] + beta_ref[...]
        x = jnp.maximum(x, 0.0)
        o_ref[...] = jnp.dot(x, iw_ref[...],
                             preferred_element_type=jnp.float32) + ib_ref[...]
    return body


def _fc_head(q, fcw, fcb, g, beta, iw, ib):
    nrois = q.shape[0]
    return pl.pallas_call(
        _make_fc_kernel(nrois),
        out_shape=jax.ShapeDtypeStruct((nrois, 1), jnp.float32),
        name="fc_iou_head",
    )(q, fcw, fcb.reshape(1, DIM), g.reshape(1, DIM), beta.reshape(1, DIM),
      iw, ib.reshape(1, 1))


def kernel(feat, proposals, conv1_w, conv1_b, bn1_g, bn1_b, conv2_w, conv2_b,
           bn2_g, bn2_b, conv3_w, conv3_b, bn3_g, bn3_b, fc_w, fc_b,
           fcbn_g, fcbn_b, iou_w, iou_b):
    ni, ns, npp = proposals.shape[0], proposals.shape[1], proposals.shape[2]
    n = ni * ns

    x0 = feat.reshape(n, DIM, HW).transpose(0, 2, 1)
    taps1 = conv1_w.transpose(3, 2, 1, 0).reshape(3, 3 * DIM, DIM)
    taps2 = conv2_w.transpose(3, 2, 1, 0).reshape(3, 3 * DIM, DIM)
    taps3 = conv3_w.transpose(3, 2, 1, 0).reshape(3, 3 * DIM, DIM)

    y1, p1 = _conv_bn(x0, taps1, conv1_b)
    s1, t1 = _bn_affine(p1, bn1_g, bn1_b, n)
    y2, p2 = _conv_bn(y1, taps2, conv2_b, s1, t1)
    s2, t2 = _bn_affine(p2, bn2_g, bn2_b, n)
    y3, p3 = _conv_bn(y2, taps3, conv3_b, s2, t2)
    s3, t3 = _bn_affine(p3, bn3_g, bn3_b, n)

    q = _pool(y3, proposals.reshape(n, npp, 4), s3, t3)

    fcw = fc_w.reshape(DIM, DIM, POOL, POOL).transpose(3, 2, 1, 0)
    fcw = fcw.reshape(POOL * POOL * DIM, DIM)
    iou = _fc_head(q.reshape(n * npp, POOL * POOL * DIM), fcw, fc_b,
                   fcbn_g, fcbn_b, iou_w.T, iou_b)
    return iou.reshape(ni, ns, npp)
